# baseline (speedup 1.0000x reference)
import functools

import jax
import jax.numpy as jnp
from jax import lax
from jax.experimental import pallas as pl
from jax.experimental.pallas import tpu as pltpu
from jax.experimental.pallas import tpu_sc as plsc

B = 4 * 8192
D = 1024
NC, NS = 2, 16
NW = NC * NS
B_PER_W = B // NW
CHUNK = 32
NCHUNK = B_PER_W // CHUNK
NBUF = 3
MAIN = NCHUNK - NBUF + 1
assert MAIN % NBUF == 0


def _gather_kernel(table_hbm, idx_hbm, out_hbm, idx_v, *rest):
    bufs = rest[:NBUF]
    gsems = rest[NBUF:2 * NBUF]
    wsems = rest[2 * NBUF:3 * NBUF]

    wid = lax.axis_index("s") * NC + lax.axis_index("c")
    base = wid * B_PER_W
    pltpu.sync_copy(idx_hbm.at[pl.ds(base, B_PER_W)], idx_v)

    def gather_cp(c, j):
        return pltpu.make_async_copy(
            table_hbm.at[idx_v.at[pl.ds(c * CHUNK, CHUNK)]], bufs[j], gsems[j]
        )

    def write_cp(c, j):
        return pltpu.make_async_copy(
            bufs[j], out_hbm.at[pl.ds(base + c * CHUNK, CHUNK)], wsems[j]
        )

    for j in range(NBUF - 1):
        gather_cp(j, j).start()

    @pl.loop(0, MAIN, step=NBUF)
    def _(g):
        for j in range(NBUF):
            cc = g + j
            gather_cp(cc, j).wait()
            write_cp(cc, j).start()
            jn = (j + NBUF - 1) % NBUF
            @pl.when(cc >= 1)
            def _():
                write_cp(cc - 1, jn).wait()
            gather_cp(cc + NBUF - 1, jn).start()

    for cc in range(MAIN, NCHUNK):
        gather_cp(cc, cc % NBUF).wait()
        write_cp(cc, cc % NBUF).start()

    for cc in range(MAIN - 1, NCHUNK):
        write_cp(cc, cc % NBUF).wait()


def kernel(position_ids, embedding_weight):
    idx = position_ids.reshape(B).astype(jnp.int32)
    mesh = plsc.VectorSubcoreMesh(core_axis_name="c", subcore_axis_name="s")
    k = functools.partial(
        pl.kernel,
        mesh=mesh,
        out_type=jax.ShapeDtypeStruct((B, D), jnp.float32),
        scratch_types=(
            [pltpu.VMEM((B_PER_W,), jnp.int32)]
            + [pltpu.VMEM((CHUNK, D), jnp.float32) for _ in range(NBUF)]
            + [pltpu.SemaphoreType.DMA for _ in range(2 * NBUF)]
        ),
    )(_gather_kernel)
    out = k(embedding_weight, idx)
    return out.reshape(4, 8192, D)
